# baseline (speedup 1.0000x reference)
import functools

import jax
import jax.numpy as jnp
from jax import lax
from jax.experimental import pallas as pl
from jax.experimental.pallas import tpu as pltpu
from jax.experimental.pallas import tpu_sc as plsc

_NUM_CORES = 2
_NUM_SUBCORES = 16
_NW = _NUM_CORES * _NUM_SUBCORES

_CHUNK = 128
_NBUF = 4

_VB = 128


@functools.cache
def _build_pack(V, D):
    grid = (V + _VB - 1) // _VB

    def pack_kernel(t_ref, o_ref):
        j = lax.broadcasted_iota(jnp.int32, (_VB, _VB // 2), 0)
        q = lax.broadcasted_iota(jnp.int32, (_VB, _VB // 2), 1)
        blk = t_ref[...]
        even = jnp.dot(blk, (j == 2 * q).astype(jnp.float32),
                       preferred_element_type=jnp.float32).T
        odd = jnp.dot(blk, (j == 2 * q + 1).astype(jnp.float32),
                      preferred_element_type=jnp.float32).T
        o_ref[...] = jnp.concatenate([even, odd], axis=1)

    return pl.pallas_call(
        pack_kernel,
        grid=(grid,),
        in_specs=[pl.BlockSpec((D, _VB), lambda g: (0, g))],
        out_specs=pl.BlockSpec((_VB // 2, 2 * D), lambda g: (g, 0)),
        out_shape=jax.ShapeDtypeStruct((V // 2, 2 * D), jnp.float32),
    )


@functools.cache
def _build_gather(V, D, S):
    mesh = plsc.VectorSubcoreMesh(core_axis_name="c", subcore_axis_name="s")

    @functools.partial(
        pl.kernel,
        out_type=jax.ShapeDtypeStruct((_NW, S, _CHUNK, 2 * D), jnp.float32),
        mesh=mesh,
        compiler_params=pltpu.CompilerParams(use_tc_tiling_on_sc=False),
        scratch_types=[
            pltpu.VMEM((S, _CHUNK), jnp.int32),
            pltpu.VMEM((_NBUF, _CHUNK, D), jnp.float32),
            [pltpu.SemaphoreType.DMA] * _NBUF,
            [pltpu.SemaphoreType.DMA] * _NBUF,
        ],
    )
    def gather_kernel(idx_hbm, table_hbm, out_hbm, idx_v, rows_v, gsems, wsems):
        wid = lax.axis_index("s") * _NUM_CORES + lax.axis_index("c")

        pltpu.sync_copy(idx_hbm.at[wid], idx_v)

        def start_gather(c, slot):
            pltpu.async_copy(table_hbm.at[idx_v.at[c]], rows_v.at[slot],
                             gsems[slot])

        def wait_gather(c, slot):
            pltpu.make_async_copy(table_hbm.at[idx_v.at[c]], rows_v.at[slot],
                                  gsems[slot]).wait()

        def start_write(c, slot):
            pltpu.async_copy(rows_v.at[slot],
                             out_hbm.at[wid, c, :, pl.ds(0, D)], wsems[slot])

        def wait_write(c, slot):
            pltpu.make_async_copy(rows_v.at[slot],
                                  out_hbm.at[wid, c, :, pl.ds(0, D)],
                                  wsems[slot]).wait()

        for b in range(_NBUF):
            start_gather(b, b)

        def body(g, _):
            for b in range(_NBUF):
                c = g * _NBUF + b
                wait_gather(c, b)
                start_write(c, b)
                wait_write(c, b)
                start_gather(c + _NBUF, b)
            return _

        lax.fori_loop(0, S // _NBUF - 1, body, None)

        for b in range(_NBUF):
            c = S - _NBUF + b
            wait_gather(c, b)
            start_write(c, b)
        for b in range(_NBUF):
            wait_write(S - _NBUF + b, b)

    return gather_kernel


def kernel(X, table):
    V, D = table.shape
    B = X.size
    assert B % (_NW * _CHUNK) == 0
    S = B // (_NW * _CHUNK)
    table_rm = _build_pack(V, D)(table.T).reshape(V, D)
    idx = X.reshape(_NW, S, _CHUNK).astype(jnp.int32)
    out = _build_gather(V, D, S)(idx, table_rm)
    return out.reshape(B, 2 * D)[:, :D].reshape(X.shape + (D,))


# speedup vs baseline: 7.2647x; 7.2647x over previous
import functools

import jax
import jax.numpy as jnp
from jax import lax
from jax.experimental import pallas as pl
from jax.experimental.pallas import tpu as pltpu
from jax.experimental.pallas import tpu_sc as plsc

_NUM_CORES = 2
_NUM_SUBCORES = 16
_NW = _NUM_CORES * _NUM_SUBCORES

_CHUNK = 128
_NBUF = 4

_VB = 128


_PW = 2048


@functools.cache
def _build_pack(V, D):
    grid = (V + 2 * _PW - 1) // (2 * _PW)
    h_pad = grid * _PW
    last = (V - 1) // _PW

    def pack_kernel(lo_ref, hi_ref, o_ref):
        o_ref[...] = jnp.concatenate([lo_ref[...].T, hi_ref[...].T], axis=1)

    return pl.pallas_call(
        pack_kernel,
        grid=(grid,),
        in_specs=[
            pl.BlockSpec((D, _PW), lambda g: (0, jnp.minimum(2 * g, last))),
            pl.BlockSpec((D, _PW),
                         lambda g: (0, jnp.minimum(2 * g + 1, last))),
        ],
        out_specs=pl.BlockSpec((_PW, 2 * D), lambda g: (g, 0)),
        out_shape=jax.ShapeDtypeStruct((h_pad, 2 * D), jnp.float32),
    )


@functools.cache
def _build_gather(V, D, S):
    mesh = plsc.VectorSubcoreMesh(core_axis_name="c", subcore_axis_name="s")

    @functools.partial(
        pl.kernel,
        out_type=jax.ShapeDtypeStruct((_NW, S, _CHUNK, 2 * D), jnp.float32),
        mesh=mesh,
        compiler_params=pltpu.CompilerParams(use_tc_tiling_on_sc=False),
        scratch_types=[
            pltpu.VMEM((S, _CHUNK), jnp.int32),
            pltpu.VMEM((_NBUF, _CHUNK, D), jnp.float32),
            [pltpu.SemaphoreType.DMA] * _NBUF,
            [pltpu.SemaphoreType.DMA] * _NBUF,
        ],
    )
    def gather_kernel(idx_hbm, table_hbm, out_hbm, idx_v, rows_v, gsems, wsems):
        wid = lax.axis_index("s") * _NUM_CORES + lax.axis_index("c")

        pltpu.sync_copy(idx_hbm.at[wid], idx_v)

        def start_gather(c, slot):
            pltpu.async_copy(table_hbm.at[idx_v.at[c]], rows_v.at[slot],
                             gsems[slot])

        def wait_gather(c, slot):
            pltpu.make_async_copy(table_hbm.at[idx_v.at[c]], rows_v.at[slot],
                                  gsems[slot]).wait()

        def start_write(c, slot):
            pltpu.async_copy(rows_v.at[slot],
                             out_hbm.at[wid, c, :, pl.ds(0, D)], wsems[slot])

        def wait_write(c, slot):
            pltpu.make_async_copy(rows_v.at[slot],
                                  out_hbm.at[wid, c, :, pl.ds(0, D)],
                                  wsems[slot]).wait()

        for b in range(_NBUF):
            start_gather(b, b)

        def body(g, _):
            for b in range(_NBUF):
                c = g * _NBUF + b
                wait_gather(c, b)
                start_write(c, b)
                wait_write(c, b)
                start_gather(c + _NBUF, b)
            return _

        lax.fori_loop(0, S // _NBUF - 1, body, None)

        for b in range(_NBUF):
            c = S - _NBUF + b
            wait_gather(c, b)
            start_write(c, b)
        for b in range(_NBUF):
            wait_write(S - _NBUF + b, b)

    return gather_kernel


def kernel(X, table):
    V, D = table.shape
    B = X.size
    assert B % (_NW * _CHUNK) == 0
    S = B // (_NW * _CHUNK)
    tT = table.T
    packed = _build_pack(V, D)(tT, tT)
    V2 = packed.shape[0] * 2
    table_rm = packed.reshape(V2, D)
    Xr = X.astype(jnp.int32)
    g = Xr // (2 * _PW)
    r = Xr % (2 * _PW)
    Xi = 2 * (_PW * g + (r & (_PW - 1))) + (r // _PW)
    idx = Xi.reshape(_NW, S, _CHUNK)
    out = _build_gather(V2, D, S)(idx, table_rm)
    return out.reshape(B, 2 * D)[:, :D].reshape(X.shape + (D,))


# speedup vs baseline: 8.0718x; 1.1111x over previous
import functools

import jax
import jax.numpy as jnp
from jax import lax
from jax.experimental import pallas as pl
from jax.experimental.pallas import tpu as pltpu
from jax.experimental.pallas import tpu_sc as plsc

_NUM_CORES = 2
_NUM_SUBCORES = 16
_NW = _NUM_CORES * _NUM_SUBCORES

_CHUNK = 128
_NBUF = 4

_VB = 128


_PW = 4096


@functools.cache
def _build_pack(V, D):
    grid = (V + 2 * _PW - 1) // (2 * _PW)
    h_pad = grid * _PW
    last = (V - 1) // _PW

    def pack_kernel(lo_ref, hi_ref, o_ref):
        o_ref[...] = jnp.concatenate([lo_ref[...].T, hi_ref[...].T], axis=1)

    return pl.pallas_call(
        pack_kernel,
        grid=(grid,),
        in_specs=[
            pl.BlockSpec((D, _PW), lambda g: (0, jnp.minimum(2 * g, last))),
            pl.BlockSpec((D, _PW),
                         lambda g: (0, jnp.minimum(2 * g + 1, last))),
        ],
        out_specs=pl.BlockSpec((_PW, 2 * D), lambda g: (g, 0)),
        out_shape=jax.ShapeDtypeStruct((h_pad, 2 * D), jnp.float32),
    )


@functools.cache
def _build_gather(V, D, S):
    mesh = plsc.VectorSubcoreMesh(core_axis_name="c", subcore_axis_name="s")

    @functools.partial(
        pl.kernel,
        out_type=jax.ShapeDtypeStruct((_NW, S, _CHUNK, 2 * D), jnp.float32),
        mesh=mesh,
        compiler_params=pltpu.CompilerParams(use_tc_tiling_on_sc=False),
        scratch_types=[
            pltpu.VMEM((S, _CHUNK), jnp.int32),
            pltpu.VMEM((_NBUF, _CHUNK, D), jnp.float32),
            [pltpu.SemaphoreType.DMA] * _NBUF,
            [pltpu.SemaphoreType.DMA] * _NBUF,
        ],
    )
    def gather_kernel(idx_hbm, table_hbm, out_hbm, idx_v, rows_v, gsems, wsems):
        wid = lax.axis_index("s") * _NUM_CORES + lax.axis_index("c")

        pltpu.sync_copy(idx_hbm.at[wid], idx_v)

        def start_gather(c, slot):
            pltpu.async_copy(table_hbm.at[idx_v.at[c]], rows_v.at[slot],
                             gsems[slot])

        def wait_gather(c, slot):
            pltpu.make_async_copy(table_hbm.at[idx_v.at[c]], rows_v.at[slot],
                                  gsems[slot]).wait()

        def start_write(c, slot):
            pltpu.async_copy(rows_v.at[slot],
                             out_hbm.at[wid, c, :, pl.ds(0, D)], wsems[slot])

        def wait_write(c, slot):
            pltpu.make_async_copy(rows_v.at[slot],
                                  out_hbm.at[wid, c, :, pl.ds(0, D)],
                                  wsems[slot]).wait()

        for b in range(_NBUF):
            start_gather(b, b)

        def body(g, _):
            for b in range(_NBUF):
                c = g * _NBUF + b
                wait_gather(c, b)
                start_write(c, b)
                wait_write(c, b)
                start_gather(c + _NBUF, b)
            return _

        lax.fori_loop(0, S // _NBUF - 1, body, None)

        for b in range(_NBUF):
            c = S - _NBUF + b
            wait_gather(c, b)
            start_write(c, b)
        for b in range(_NBUF):
            wait_write(S - _NBUF + b, b)

    return gather_kernel


def kernel(X, table):
    V, D = table.shape
    B = X.size
    assert B % (_NW * _CHUNK) == 0
    S = B // (_NW * _CHUNK)
    tT = table.T
    packed = _build_pack(V, D)(tT, tT)
    V2 = packed.shape[0] * 2
    table_rm = packed.reshape(V2, D)
    Xr = X.astype(jnp.int32)
    g = Xr // (2 * _PW)
    r = Xr % (2 * _PW)
    Xi = 2 * (_PW * g + (r & (_PW - 1))) + (r // _PW)
    idx = Xi.reshape(_NW, S, _CHUNK)
    out = _build_gather(V2, D, S)(idx, table_rm)
    return out.reshape(B, 2 * D)[:, :D].reshape(X.shape + (D,))
